# baseline (speedup 1.0000x reference)
import functools

import jax
import jax.numpy as jnp
from jax import lax
from jax.experimental import pallas as pl
from jax.experimental.pallas import tpu as pltpu
from jax.experimental.pallas import tpu_sc as plsc

VOCAB = 1000000
D = 64
H = 32
B = 4096
SEQ = 200

NC = 2
NS = 16
NW = NC * NS
RW = B // NW
CH = 100
NCHUNK = SEQ // CH
LANES = 16


def _sc_pool_body(tok_hbm, table_hbm, out_hbm, tok_v, buf_v, pool_v, sem):
    wid = lax.axis_index("s") * NC + lax.axis_index("c")
    cbase = wid * (NCHUNK * RW)
    pltpu.sync_copy(tok_hbm.at[pl.ds(cbase, NCHUNK * RW)], tok_v)

    zero = jnp.zeros((LANES,), jnp.float32)

    def do_row(b, carry):
        cps = []
        for s in range(NCHUNK):
            cps.append(
                pltpu.async_copy(
                    table_hbm.at[tok_v.at[NCHUNK * b + s]], buf_v.at[s], sem
                )
            )
        for cp in cps:
            cp.wait()

        def red(r, accs):
            accs = list(accs)
            for s in range(NCHUNK):
                for k in range(D // LANES):
                    accs[k] = accs[k] + buf_v[s, r, pl.ds(LANES * k, LANES)]
            return tuple(accs)

        accs = lax.fori_loop(0, CH, red, (zero,) * (D // LANES))
        for k in range(D // LANES):
            pool_v[b, pl.ds(LANES * k, LANES)] = accs[k]
        return carry

    lax.fori_loop(0, RW, do_row, 0)
    pltpu.sync_copy(pool_v, out_hbm.at[pl.ds(wid * RW, RW)])


@functools.partial(jax.jit, static_argnames=())
def _sc_pool(tok2, table):
    mesh = plsc.VectorSubcoreMesh(
        core_axis_name="c", subcore_axis_name="s", num_cores=NC, num_subcores=NS
    )
    return pl.kernel(
        _sc_pool_body,
        out_type=jax.ShapeDtypeStruct((B, D), jnp.float32),
        mesh=mesh,
        scratch_types=[
            pltpu.VMEM((NCHUNK * RW, CH), jnp.int32),
            pltpu.VMEM((NCHUNK, CH, D), jnp.float32),
            pltpu.VMEM((RW, D), jnp.float32),
            pltpu.SemaphoreType.DMA,
        ],
        compiler_params=pltpu.CompilerParams(use_tc_tiling_on_sc=False),
    )(tok2, table)


def _mlp_body(pool_ref, w1_ref, b1_ref, w2_ref, b2_ref, out_ref):
    pooled = pool_ref[...] * (1.0 / SEQ)
    h = jnp.dot(pooled, w1_ref[...], preferred_element_type=jnp.float32)
    h = jnp.maximum(h + b1_ref[...], 0.0)
    out_ref[...] = jnp.sum(h * w2_ref[...], axis=1, keepdims=True) + b2_ref[...]


@jax.jit
def _mlp(pooled, W1, b1r, W2r, b2r):
    return pl.pallas_call(
        _mlp_body,
        out_shape=jax.ShapeDtypeStruct((B, 1), jnp.float32),
    )(pooled, W1, b1r, W2r, b2r)


def kernel(tokens, table, W1, b1, W2, b2):
    tok2 = tokens.reshape(NCHUNK * B, CH).astype(jnp.int32)
    pooled = _sc_pool(tok2, table)
    out = _mlp(
        pooled,
        W1,
        b1.reshape(1, H),
        W2.reshape(1, H),
        b2.reshape(1, 1),
    )
    return out[:, 0]


# speedup vs baseline: 1.1321x; 1.1321x over previous
import functools

import jax
import jax.numpy as jnp
from jax import lax
from jax.experimental import pallas as pl
from jax.experimental.pallas import tpu as pltpu
from jax.experimental.pallas import tpu_sc as plsc

VOCAB = 1000000
D = 64
H = 32
B = 4096
SEQ = 200

NC = 2
NS = 16
NW = NC * NS
RW = B // NW
CH = 100
NCHUNK = SEQ // CH
LANES = 16


RUNROLL = 4


def _sc_pool_body(tok_hbm, table_hbm, out_hbm, tok_v, buf_v, pool_v, sem0, sem1):
    wid = lax.axis_index("s") * NC + lax.axis_index("c")
    cbase = wid * (NCHUNK * RW)
    pltpu.sync_copy(tok_hbm.at[pl.ds(cbase, NCHUNK * RW)], tok_v)

    zero = jnp.zeros((LANES,), jnp.float32)
    sems = (sem0, sem1)

    def issue(b, slot):
        for s in range(NCHUNK):
            pltpu.async_copy(
                table_hbm.at[tok_v.at[NCHUNK * b + s]],
                buf_v.at[slot, s],
                sems[slot],
            )

    def wait(b, slot):
        for s in range(NCHUNK):
            pltpu.make_async_copy(
                table_hbm.at[tok_v.at[NCHUNK * b + s]],
                buf_v.at[slot, s],
                sems[slot],
            ).wait()

    def reduce_row(b, slot):
        def red(r, accs):
            accs = list(accs)
            for u in range(RUNROLL):
                for s in range(NCHUNK):
                    for k in range(D // LANES):
                        accs[k] = accs[k] + buf_v[
                            slot, s, r + u, pl.ds(LANES * k, LANES)
                        ]
            return tuple(accs)

        accs = lax.fori_loop(
            0, CH // RUNROLL,
            lambda i, a: red(i * RUNROLL, a),
            (zero,) * (D // LANES),
        )
        for k in range(D // LANES):
            pool_v[b, pl.ds(LANES * k, LANES)] = accs[k]

    issue(0, 0)

    def do_pair(i, carry):
        b0 = 2 * i
        b1 = 2 * i + 1
        issue(b1, 1)
        wait(b0, 0)
        reduce_row(b0, 0)

        @pl.when(i < RW // 2 - 1)
        def _():
            issue(b0 + 2, 0)

        wait(b1, 1)
        reduce_row(b1, 1)
        return carry

    lax.fori_loop(0, RW // 2, do_pair, 0)
    pltpu.sync_copy(pool_v, out_hbm.at[pl.ds(wid * RW, RW)])


@functools.partial(jax.jit, static_argnames=())
def _sc_pool(tok2, table):
    mesh = plsc.VectorSubcoreMesh(
        core_axis_name="c", subcore_axis_name="s", num_cores=NC, num_subcores=NS
    )
    return pl.kernel(
        _sc_pool_body,
        out_type=jax.ShapeDtypeStruct((B, D), jnp.float32),
        mesh=mesh,
        scratch_types=[
            pltpu.VMEM((NCHUNK * RW, CH), jnp.int32),
            pltpu.VMEM((2, NCHUNK, CH, D), jnp.float32),
            pltpu.VMEM((RW, D), jnp.float32),
            pltpu.SemaphoreType.DMA,
            pltpu.SemaphoreType.DMA,
        ],
        compiler_params=pltpu.CompilerParams(use_tc_tiling_on_sc=False),
    )(tok2, table)


def _mlp_body(pool_ref, w1_ref, b1_ref, w2_ref, b2_ref, out_ref):
    pooled = pool_ref[...] * (1.0 / SEQ)
    h = jnp.dot(pooled, w1_ref[...], preferred_element_type=jnp.float32)
    h = jnp.maximum(h + b1_ref[...], 0.0)
    out_ref[...] = jnp.sum(h * w2_ref[...], axis=1, keepdims=True) + b2_ref[...]


@jax.jit
def _mlp(pooled, W1, b1r, W2r, b2r):
    return pl.pallas_call(
        _mlp_body,
        out_shape=jax.ShapeDtypeStruct((B, 1), jnp.float32),
    )(pooled, W1, b1r, W2r, b2r)


def kernel(tokens, table, W1, b1, W2, b2):
    tok2 = tokens.reshape(NCHUNK * B, CH).astype(jnp.int32)
    pooled = _sc_pool(tok2, table)
    out = _mlp(
        pooled,
        W1,
        b1.reshape(1, H),
        W2.reshape(1, H),
        b2.reshape(1, 1),
    )
    return out[:, 0]
